# Initial kernel scaffold; baseline (speedup 1.0000x reference)
import jax
import jax.numpy as jnp
from jax.experimental import pallas as pl


def kernel(class_logits, pred_bboxes):
    raise NotImplementedError("write your pallas kernel here")


# speedup vs baseline: 2.2029x; 2.2029x over previous
import jax
import jax.numpy as jnp
from jax.experimental import pallas as pl


def _feat_body(cl_ref, s_ref, sb_ref, sc_ref):
    x = cl_ref[...]
    s = jax.nn.sigmoid(x)
    s_ref[...] = s
    sb_ref[...] = jnp.minimum(1.0 - s, s)
    ms = jnp.max(s, axis=-1, keepdims=True)
    lp = jnp.log(ms + 1e-8)
    ln = jnp.log(1.0 - ms + 1e-8)
    sc_ref[...] = jnp.concatenate([lp, ln, ms], axis=-1)


def _expand_body(bb_ref, s_ref, sb_ref, sc_ref, out_ref):
    x = pl.program_id(1)
    N = s_ref.shape[1]
    P = N - 1

    rid = jax.lax.broadcasted_iota(jnp.int32, (P, 1), 0)
    cond = rid < x

    def sel(ref):
        return jnp.where(cond, ref[0, : N - 1, :], ref[0, 1:N, :])

    def bc(ref):
        row = ref[0, pl.ds(x, 1), :]
        return jnp.broadcast_to(row, (P, ref.shape[2]))

    bbY = sel(bb_ref)
    sY = sel(s_ref)
    sbY = sel(sb_ref)
    scY = sel(sc_ref)
    bbX = bc(bb_ref)
    sX = bc(s_ref)
    sbX = bc(sb_ref)
    scX = bc(sc_ref)

    out_ref[0, 0] = jnp.concatenate(
        [
            bbX, bbY, sX, sY, sbX, sbY,
            scX[:, 0:1], scY[:, 0:1],
            scX[:, 1:2], scY[:, 1:2],
            scX[:, 2:3], scY[:, 2:3],
        ],
        axis=1,
    )


def kernel(class_logits, pred_bboxes):
    B, N, C = class_logits.shape
    W = 8 + 4 * C + 6

    s, sb, sc = pl.pallas_call(
        _feat_body,
        grid=(B,),
        in_specs=[pl.BlockSpec((1, N, C), lambda b: (b, 0, 0))],
        out_specs=[
            pl.BlockSpec((1, N, C), lambda b: (b, 0, 0)),
            pl.BlockSpec((1, N, C), lambda b: (b, 0, 0)),
            pl.BlockSpec((1, N, 3), lambda b: (b, 0, 0)),
        ],
        out_shape=[
            jax.ShapeDtypeStruct((B, N, C), jnp.float32),
            jax.ShapeDtypeStruct((B, N, C), jnp.float32),
            jax.ShapeDtypeStruct((B, N, 3), jnp.float32),
        ],
    )(class_logits)

    out4 = pl.pallas_call(
        _expand_body,
        grid=(B, N),
        in_specs=[
            pl.BlockSpec((1, N, 4), lambda b, x: (b, 0, 0)),
            pl.BlockSpec((1, N, C), lambda b, x: (b, 0, 0)),
            pl.BlockSpec((1, N, C), lambda b, x: (b, 0, 0)),
            pl.BlockSpec((1, N, 3), lambda b, x: (b, 0, 0)),
        ],
        out_specs=pl.BlockSpec((1, 1, N - 1, W), lambda b, x: (b, x, 0, 0)),
        out_shape=jax.ShapeDtypeStruct((B, N, N - 1, W), jnp.float32),
    )(pred_bboxes, s, sb, sc)

    return out4.reshape(B, N * (N - 1), W)


# speedup vs baseline: 9.7759x; 4.4377x over previous
import jax
import jax.numpy as jnp
from jax.experimental import pallas as pl

_XG = 8


def _feat_body(cl_ref, bb_ref, fx_ref, fy_ref):
    x = cl_ref[0]
    bb = bb_ref[0]
    s = jax.nn.sigmoid(x)
    sb = jnp.minimum(1.0 - s, s)
    ms = jnp.max(s, axis=-1, keepdims=True)
    lp = jnp.log(ms + 1e-8)
    ln = jnp.log(1.0 - ms + 1e-8)
    N, C = x.shape
    z4 = jnp.zeros((N, 4), jnp.float32)
    zc = jnp.zeros((N, C), jnp.float32)
    z1 = jnp.zeros((N, 1), jnp.float32)
    fx_ref[0] = jnp.concatenate(
        [bb, z4, s, zc, sb, zc, lp, z1, ln, z1, ms, z1], axis=1)
    fy_ref[0] = jnp.concatenate(
        [z4, bb, zc, s, zc, sb, z1, lp, z1, ln, z1, ms], axis=1)


def _expand_body(fx_ref, fy_ref, out_ref):
    k = pl.program_id(1)
    N = fx_ref.shape[1]
    P = N - 1
    W = fx_ref.shape[2]
    fya = fy_ref[0, :P, :]
    fyb = fy_ref[0, 1:, :]
    rid = jax.lax.broadcasted_iota(jnp.int32, (P, 1), 0)
    for j in range(_XG):
        x = k * _XG + j
        fx_row = jnp.broadcast_to(fx_ref[0, pl.ds(x, 1), :], (P, W))
        out_ref[0, pl.ds(j * P, P), :] = jnp.where(rid < x, fya, fyb) + fx_row


def kernel(class_logits, pred_bboxes):
    B, N, C = class_logits.shape
    W = 8 + 4 * C + 6
    P = N * (N - 1)

    fx, fy = pl.pallas_call(
        _feat_body,
        grid=(B,),
        in_specs=[
            pl.BlockSpec((1, N, C), lambda b: (b, 0, 0)),
            pl.BlockSpec((1, N, 4), lambda b: (b, 0, 0)),
        ],
        out_specs=[
            pl.BlockSpec((1, N, W), lambda b: (b, 0, 0)),
            pl.BlockSpec((1, N, W), lambda b: (b, 0, 0)),
        ],
        out_shape=[
            jax.ShapeDtypeStruct((B, N, W), jnp.float32),
            jax.ShapeDtypeStruct((B, N, W), jnp.float32),
        ],
    )(class_logits, pred_bboxes)

    rows = _XG * (N - 1)
    grid_k = (N + _XG - 1) // _XG
    out = pl.pallas_call(
        _expand_body,
        grid=(B, grid_k),
        in_specs=[
            pl.BlockSpec((1, N, W), lambda b, k: (b, 0, 0)),
            pl.BlockSpec((1, N, W), lambda b, k: (b, 0, 0)),
        ],
        out_specs=pl.BlockSpec((1, rows, W), lambda b, k: (b, k, 0)),
        out_shape=jax.ShapeDtypeStruct((B, P, W), jnp.float32),
    )(fx, fy)

    return out


# speedup vs baseline: 10.0947x; 1.0326x over previous
import jax
import jax.numpy as jnp
from jax.experimental import pallas as pl

_XG = 16


def _feat_body(cl_ref, bb_ref, fx_ref, fy_ref):
    x = cl_ref[0]
    bb = bb_ref[0]
    s = jax.nn.sigmoid(x)
    sb = jnp.minimum(1.0 - s, s)
    ms = jnp.max(s, axis=-1, keepdims=True)
    lp = jnp.log(ms + 1e-8)
    ln = jnp.log(1.0 - ms + 1e-8)
    N, C = x.shape
    z4 = jnp.zeros((N, 4), jnp.float32)
    zc = jnp.zeros((N, C), jnp.float32)
    z1 = jnp.zeros((N, 1), jnp.float32)
    fx_ref[0] = jnp.concatenate(
        [bb, z4, s, zc, sb, zc, lp, z1, ln, z1, ms, z1], axis=1)
    fy_ref[0] = jnp.concatenate(
        [z4, bb, zc, s, zc, sb, z1, lp, z1, ln, z1, ms], axis=1)


def _expand_body(fx_ref, fy_ref, out_ref):
    k = pl.program_id(1)
    N = fx_ref.shape[1]
    P = N - 1
    W = fx_ref.shape[2]
    fya = fy_ref[0, :P, :]
    fyb = fy_ref[0, 1:, :]
    rid = jax.lax.broadcasted_iota(jnp.int32, (P, 1), 0)
    for j in range(_XG):
        x = k * _XG + j
        fx_row = jnp.broadcast_to(fx_ref[0, pl.ds(x, 1), :], (P, W))
        out_ref[0, pl.ds(j * P, P), :] = jnp.where(rid < x, fya, fyb) + fx_row


def kernel(class_logits, pred_bboxes):
    B, N, C = class_logits.shape
    W = 8 + 4 * C + 6
    P = N * (N - 1)

    fx, fy = pl.pallas_call(
        _feat_body,
        grid=(B,),
        in_specs=[
            pl.BlockSpec((1, N, C), lambda b: (b, 0, 0)),
            pl.BlockSpec((1, N, 4), lambda b: (b, 0, 0)),
        ],
        out_specs=[
            pl.BlockSpec((1, N, W), lambda b: (b, 0, 0)),
            pl.BlockSpec((1, N, W), lambda b: (b, 0, 0)),
        ],
        out_shape=[
            jax.ShapeDtypeStruct((B, N, W), jnp.float32),
            jax.ShapeDtypeStruct((B, N, W), jnp.float32),
        ],
    )(class_logits, pred_bboxes)

    rows = _XG * (N - 1)
    grid_k = (N + _XG - 1) // _XG
    out = pl.pallas_call(
        _expand_body,
        grid=(B, grid_k),
        in_specs=[
            pl.BlockSpec((1, N, W), lambda b, k: (b, 0, 0)),
            pl.BlockSpec((1, N, W), lambda b, k: (b, 0, 0)),
        ],
        out_specs=pl.BlockSpec((1, rows, W), lambda b, k: (b, k, 0)),
        out_shape=jax.ShapeDtypeStruct((B, P, W), jnp.float32),
    )(fx, fy)

    return out
